# Initial kernel scaffold; baseline (speedup 1.0000x reference)
import jax
import jax.numpy as jnp
from jax.experimental import pallas as pl


def kernel(court_preds, score_preds, court_targs, score_targs):
    raise NotImplementedError("write your pallas kernel here")


# speedup vs baseline: 10.4608x; 10.4608x over previous
import functools

import numpy as np
import jax
import jax.numpy as jnp
from jax import lax
from jax.experimental import pallas as pl
from jax.experimental.pallas import tpu as pltpu
from jax.experimental.pallas import tpu_sc as plsc

B = 32
N = 224 * 224
LANES = 16
CHUNKS = N // LANES
UNROLL = 8
OUTER = CHUNKS // UNROLL
MIN_I32 = -2147483648


def _sortable(v):
    return v ^ ((v >> 31) & 0x7FFFFFFF)


def _sc_body(cp_hbm, ct_hbm, out_hbm, a_ref, b_ref, res_ref):
    wid = lax.axis_index("s") * 2 + lax.axis_index("c")
    pltpu.sync_copy(cp_hbm.at[wid], a_ref)
    pltpu.sync_copy(ct_hbm.at[wid], b_ref)

    iota16 = lax.iota(jnp.int32, LANES)
    zero_i = jnp.zeros((LANES,), jnp.int32)

    def p0(i, npos_vec):
        base = i * UNROLL
        for u in range(UNROLL):
            ctv = b_ref[base + u]
            v = plsc.bitcast(a_ref[base + u], jnp.int32)
            a_ref[base + u] = plsc.bitcast(_sortable(v), jnp.float32)
            npos_vec = npos_vec + jnp.where(ctv > 0.1, 1, 0)
        return npos_vec

    npos_vec = lax.fori_loop(0, OUTER, p0, zero_i)
    num_pos = jnp.sum(npos_vec)
    k = jnp.minimum(3 * num_pos, N - 1)

    def count_ge(cand_signed):
        def body(i, cnt_vec):
            base = i * UNROLL
            for u in range(UNROLL):
                s = plsc.bitcast(a_ref[base + u], jnp.int32)
                cnt_vec = cnt_vec + jnp.where(s >= cand_signed, 1, 0)
            return cnt_vec
        return jnp.sum(lax.fori_loop(0, OUTER, body, zero_i))

    def sbody(it, t_off):
        cand = t_off | jnp.left_shift(1, 31 - it)
        cnt = count_ge(cand ^ MIN_I32)
        return jnp.where(cnt >= k, cand, t_off)

    t_off = lax.fori_loop(0, 32, sbody, np.int32(0))
    t = t_off ^ MIN_I32

    def cpass(i, carry):
        gt_vec, ge_vec = carry
        base = i * UNROLL
        for u in range(UNROLL):
            s = plsc.bitcast(a_ref[base + u], jnp.int32)
            gt_vec = gt_vec + jnp.where(s > t, 1, 0)
            ge_vec = ge_vec + jnp.where(s >= t, 1, 0)
        return gt_vec, ge_vec

    gt_vec, ge_vec = lax.fori_loop(0, OUTER, cpass, (zero_i, zero_i))
    cnt_gt = jnp.sum(gt_vec)
    cnt_ge = jnp.sum(ge_vec)
    m = k - cnt_gt
    n_ties = cnt_ge - cnt_gt

    def idx_search(_):
        def count_tie_lt(cand):
            def body(i, cnt_vec):
                base = i * UNROLL
                for u in range(UNROLL):
                    s = plsc.bitcast(a_ref[base + u], jnp.int32)
                    idx = (base + u) * LANES + iota16
                    cnt_vec = cnt_vec + jnp.where((s == t) & (idx < cand), 1, 0)
                return cnt_vec
            return jnp.sum(lax.fori_loop(0, OUTER, body, zero_i))

        def ibody(it, cut):
            cand = cut | jnp.left_shift(1, 16 - it)
            return jnp.where(count_tie_lt(cand) <= m, cand, cut)

        return lax.fori_loop(0, 17, ibody, np.int32(0))

    cut = lax.cond(n_ties == m, lambda _: np.int32(131072), idx_search, 0)

    def fpass(i, carry):
        sq_vec, keep_vec = carry
        base = i * UNROLL
        for u in range(UNROLL):
            s = plsc.bitcast(a_ref[base + u], jnp.int32)
            cpv = plsc.bitcast(_sortable(s), jnp.float32)
            ctv = b_ref[base + u]
            idx = (base + u) * LANES + iota16
            keep = (ctv > 0.1) | (s > t) | ((s == t) & (idx < cut))
            d = cpv - ctv
            sq_vec = sq_vec + jnp.where(keep, d * d, 0.0)
            keep_vec = keep_vec + jnp.where(keep, 1, 0)
        return sq_vec, keep_vec

    sq_vec, keep_vec = lax.fori_loop(
        0, OUTER, fpass, (jnp.zeros((LANES,), jnp.float32), zero_i))
    sq_sum = jnp.sum(sq_vec)
    n_keep = jnp.sum(keep_vec).astype(jnp.float32)

    res = jnp.where(iota16 == 0, sq_sum,
                    jnp.where(iota16 == 1, n_keep, 0.0))
    res_ref[...] = res
    pltpu.sync_copy(res_ref, out_hbm.at[wid])


@functools.partial(jax.jit, static_argnums=())
def _sc_partials(cp, ct):
    mesh = plsc.VectorSubcoreMesh(core_axis_name="c", subcore_axis_name="s")
    f = functools.partial(
        pl.kernel,
        mesh=mesh,
        compiler_params=pltpu.CompilerParams(
            needs_layout_passes=False, use_tc_tiling_on_sc=False),
        out_type=jax.ShapeDtypeStruct((B, LANES), jnp.float32),
        scratch_types=[
            pltpu.VMEM((CHUNKS, LANES), jnp.float32),
            pltpu.VMEM((CHUNKS, LANES), jnp.float32),
            pltpu.VMEM((LANES,), jnp.float32),
        ],
    )(_sc_body)
    return f(cp, ct)


def _tc_reduce_body(part_ref, sp_ref, st_ref, out_ref):
    p = part_ref[...]
    lane = lax.broadcasted_iota(jnp.int32, p.shape, 1)
    sq_sum = jnp.sum(jnp.where(lane == 0, p, 0.0))
    n_keep = jnp.sum(jnp.where(lane == 1, p, 0.0))
    court = sq_sum / jnp.maximum(n_keep, 1.0)
    d = sp_ref[...] - st_ref[...]
    score = jnp.sum(d * d) / float(B * 8)
    out_ref[0, 0] = court + score


def _tc_reduce(partials, sp, st):
    return pl.pallas_call(
        _tc_reduce_body,
        out_shape=jax.ShapeDtypeStruct((1, 1), jnp.float32),
        out_specs=pl.BlockSpec(memory_space=pltpu.SMEM),
    )(partials, sp, st)


def kernel(court_preds, score_preds, court_targs, score_targs):
    cp = court_preds.reshape(B, CHUNKS, LANES)
    ct = court_targs.reshape(B, CHUNKS, LANES)
    partials = _sc_partials(cp, ct)
    out = _tc_reduce(partials, score_preds, score_targs)
    return out[0, 0]


# speedup vs baseline: 18.5924x; 1.7773x over previous
import functools

import numpy as np
import jax
import jax.numpy as jnp
from jax import lax
from jax.experimental import pallas as pl
from jax.experimental.pallas import tpu as pltpu
from jax.experimental.pallas import tpu_sc as plsc

B = 32
N = 224 * 224
LANES = 16
ROWLEN = 128
ROWS = N // ROWLEN
SUB = ROWLEN // LANES
MIN_I32 = -2147483648


def _sortable(v):
    return v ^ ((v >> 31) & 0x7FFFFFFF)


def _sc_body(cp_hbm, ct_hbm, out_hbm, a_ref, b_ref, res_ref):
    wid = lax.axis_index("s") * 2 + lax.axis_index("c")
    pltpu.sync_copy(cp_hbm.at[wid], a_ref)
    pltpu.sync_copy(ct_hbm.at[wid], b_ref)

    iota16 = lax.iota(jnp.int32, LANES)
    zero_i = jnp.zeros((LANES,), jnp.int32)

    def p0(i, npos_vec):
        for u in range(SUB):
            ctv = b_ref[i, pl.ds(u * LANES, LANES)]
            v = plsc.bitcast(a_ref[i, pl.ds(u * LANES, LANES)], jnp.int32)
            a_ref[i, pl.ds(u * LANES, LANES)] = plsc.bitcast(
                _sortable(v), jnp.float32)
            npos_vec = npos_vec + jnp.where(ctv > 0.1, 1, 0)
        return npos_vec

    npos_vec = lax.fori_loop(0, ROWS, p0, zero_i)
    num_pos = jnp.sum(npos_vec)
    k = jnp.minimum(3 * num_pos, N - 1)

    def count_ge(cand_signed):
        def body(i, cnt_vec):
            for u in range(SUB):
                s = plsc.bitcast(a_ref[i, pl.ds(u * LANES, LANES)], jnp.int32)
                cnt_vec = cnt_vec + jnp.where(s >= cand_signed, 1, 0)
            return cnt_vec
        return jnp.sum(lax.fori_loop(0, ROWS, body, zero_i))

    def sbody(it, t_off):
        cand = t_off | jnp.left_shift(1, 31 - it)
        cnt = count_ge(cand ^ MIN_I32)
        return jnp.where(cnt >= k, cand, t_off)

    t_off = lax.fori_loop(0, 32, sbody, np.int32(0))
    t = t_off ^ MIN_I32

    def cpass(i, carry):
        gt_vec, ge_vec = carry
        for u in range(SUB):
            s = plsc.bitcast(a_ref[i, pl.ds(u * LANES, LANES)], jnp.int32)
            gt_vec = gt_vec + jnp.where(s > t, 1, 0)
            ge_vec = ge_vec + jnp.where(s >= t, 1, 0)
        return gt_vec, ge_vec

    gt_vec, ge_vec = lax.fori_loop(0, ROWS, cpass, (zero_i, zero_i))
    cnt_gt = jnp.sum(gt_vec)
    cnt_ge = jnp.sum(ge_vec)
    m = k - cnt_gt
    n_ties = cnt_ge - cnt_gt

    def idx_search(_):
        def count_tie_lt(cand):
            def body(i, cnt_vec):
                for u in range(SUB):
                    s = plsc.bitcast(
                        a_ref[i, pl.ds(u * LANES, LANES)], jnp.int32)
                    idx = i * ROWLEN + u * LANES + iota16
                    cnt_vec = cnt_vec + jnp.where((s == t) & (idx < cand), 1, 0)
                return cnt_vec
            return jnp.sum(lax.fori_loop(0, ROWS, body, zero_i))

        def ibody(it, cut):
            cand = cut | jnp.left_shift(1, 16 - it)
            return jnp.where(count_tie_lt(cand) <= m, cand, cut)

        return lax.fori_loop(0, 17, ibody, np.int32(0))

    cut = lax.cond(n_ties == m, lambda _: np.int32(131072), idx_search, 0)

    def fpass(i, carry):
        sq_vec, keep_vec = carry
        for u in range(SUB):
            s = plsc.bitcast(a_ref[i, pl.ds(u * LANES, LANES)], jnp.int32)
            cpv = plsc.bitcast(_sortable(s), jnp.float32)
            ctv = b_ref[i, pl.ds(u * LANES, LANES)]
            idx = i * ROWLEN + u * LANES + iota16
            keep = (ctv > 0.1) | (s > t) | ((s == t) & (idx < cut))
            d = cpv - ctv
            sq_vec = sq_vec + jnp.where(keep, d * d, 0.0)
            keep_vec = keep_vec + jnp.where(keep, 1, 0)
        return sq_vec, keep_vec

    sq_vec, keep_vec = lax.fori_loop(
        0, ROWS, fpass, (jnp.zeros((LANES,), jnp.float32), zero_i))
    sq_sum = jnp.sum(sq_vec)
    n_keep = jnp.sum(keep_vec).astype(jnp.float32)

    res = jnp.where(iota16 == 0, sq_sum,
                    jnp.where(iota16 == 1, n_keep, 0.0))
    res_ref[...] = res
    pltpu.sync_copy(res_ref, out_hbm.at[wid])


@functools.partial(jax.jit, static_argnums=())
def _sc_partials(cp, ct):
    mesh = plsc.VectorSubcoreMesh(core_axis_name="c", subcore_axis_name="s")
    f = functools.partial(
        pl.kernel,
        mesh=mesh,
        compiler_params=pltpu.CompilerParams(
            needs_layout_passes=False, use_tc_tiling_on_sc=False),
        out_type=jax.ShapeDtypeStruct((B, LANES), jnp.float32),
        scratch_types=[
            pltpu.VMEM((ROWS, ROWLEN), jnp.float32),
            pltpu.VMEM((ROWS, ROWLEN), jnp.float32),
            pltpu.VMEM((LANES,), jnp.float32),
        ],
    )(_sc_body)
    return f(cp, ct)


def _tc_reduce_body(part_ref, sp_ref, st_ref, out_ref):
    p = part_ref[...]
    lane = lax.broadcasted_iota(jnp.int32, p.shape, 1)
    sq_sum = jnp.sum(jnp.where(lane == 0, p, 0.0))
    n_keep = jnp.sum(jnp.where(lane == 1, p, 0.0))
    court = sq_sum / jnp.maximum(n_keep, 1.0)
    d = sp_ref[...] - st_ref[...]
    score = jnp.sum(d * d) / float(B * 8)
    out_ref[0, 0] = court + score


def _tc_reduce(partials, sp, st):
    return pl.pallas_call(
        _tc_reduce_body,
        out_shape=jax.ShapeDtypeStruct((1, 1), jnp.float32),
        out_specs=pl.BlockSpec(memory_space=pltpu.SMEM),
    )(partials, sp, st)


def kernel(court_preds, score_preds, court_targs, score_targs):
    cp = court_preds.reshape(B, ROWS, ROWLEN)
    ct = court_targs.reshape(B, ROWS, ROWLEN)
    partials = _sc_partials(cp, ct)
    out = _tc_reduce(partials, score_preds, score_targs)
    return out[0, 0]
